# initial kernel scaffold (unmeasured)
import jax
import jax.numpy as jnp
from jax import lax
from jax.experimental import pallas as pl
from jax.experimental.pallas import tpu as pltpu


def kernel(
    x,
):
    def body(*refs):
        pass

    out_shape = jax.ShapeDtypeStruct(..., jnp.float32)
    return pl.pallas_call(body, out_shape=out_shape)(...)


# baseline (device time: 103600 ns/iter reference)
import jax
import jax.numpy as jnp
from jax import lax
from jax.experimental import pallas as pl
from jax.experimental.pallas import tpu as pltpu

N_DEV = 16


def kernel(x):
    m_per, n = x.shape
    chunk = m_per // N_DEV

    def rows(c):
        return pl.ds(c * chunk, chunk)

    def body(x_ref, out_ref, rs_buf, ag_buf,
             rs_send_sems, rs_recv_sems, ag_send_sems, ag_recv_sems):
        my = lax.axis_index("i")
        left = (my + N_DEV - 1) % N_DEV
        right = (my + 1) % N_DEV

        barrier_sem = pltpu.get_barrier_semaphore()
        for nbr in (left, right):
            pl.semaphore_signal(
                barrier_sem, inc=1,
                device_id=(nbr,), device_id_type=pl.DeviceIdType.MESH,
            )
        pl.semaphore_wait(barrier_sem, 2)

        for s in range(N_DEV - 1):
            if s == 0:
                src = x_ref.at[rows(my)]
            else:
                c = (my + N_DEV - s) % N_DEV
                rs_buf[s - 1, :, :] = rs_buf[s - 1, :, :] + x_ref[rows(c), :]
                src = rs_buf.at[s - 1]
            rdma = pltpu.make_async_remote_copy(
                src_ref=src,
                dst_ref=rs_buf.at[s],
                send_sem=rs_send_sems.at[s],
                recv_sem=rs_recv_sems.at[s],
                device_id=(right,),
                device_id_type=pl.DeviceIdType.MESH,
            )
            rdma.start()
            rdma.wait()

        r = (my + 1) % N_DEV
        out_ref[rows(r), :] = rs_buf[N_DEV - 2, :, :] + x_ref[rows(r), :]

        for t in range(N_DEV - 1):
            if t == 0:
                src = out_ref.at[rows(r)]
            else:
                src = ag_buf.at[t - 1]
            rdma = pltpu.make_async_remote_copy(
                src_ref=src,
                dst_ref=ag_buf.at[t],
                send_sem=ag_send_sems.at[t],
                recv_sem=ag_recv_sems.at[t],
                device_id=(right,),
                device_id_type=pl.DeviceIdType.MESH,
            )
            rdma.start()
            rdma.wait()
            c = (my + N_DEV - t) % N_DEV
            out_ref[rows(c), :] = ag_buf[t, :, :]

    return pl.pallas_call(
        body,
        out_shape=jax.ShapeDtypeStruct((m_per, n), x.dtype),
        in_specs=[pl.BlockSpec(memory_space=pltpu.VMEM)],
        out_specs=pl.BlockSpec(memory_space=pltpu.VMEM),
        scratch_shapes=[
            pltpu.VMEM((N_DEV - 1, chunk, n), x.dtype),
            pltpu.VMEM((N_DEV - 1, chunk, n), x.dtype),
            pltpu.SemaphoreType.DMA((N_DEV - 1,)),
            pltpu.SemaphoreType.DMA((N_DEV - 1,)),
            pltpu.SemaphoreType.DMA((N_DEV - 1,)),
            pltpu.SemaphoreType.DMA((N_DEV - 1,)),
        ],
        compiler_params=pltpu.CompilerParams(collective_id=0),
    )(x)


# device time: 64660 ns/iter; 1.6022x vs baseline; 1.6022x over previous
import jax
import jax.numpy as jnp
from jax import lax
from jax.experimental import pallas as pl
from jax.experimental.pallas import tpu as pltpu

N_DEV = 16
MASKS = (1, 3, 4, 8)


def _keep_bits(my):
    return [
        (jnp.bitwise_xor(my, my >> 1)) & 1,
        (my >> 1) & 1,
        (my >> 2) & 1,
        (my >> 3) & 1,
    ]


def kernel(x):
    m_per, n = x.shape

    def body(x_ref, out_ref, rs0, rs1, rs2, rs3, send_sems, recv_sems):
        my = lax.axis_index("i")
        rs_bufs = [rs0, rs1, rs2, rs3]

        barrier_sem = pltpu.get_barrier_semaphore()
        for mask in MASKS:
            pl.semaphore_signal(
                barrier_sem, inc=1,
                device_id=(jnp.bitwise_xor(my, mask),),
                device_id_type=pl.DeviceIdType.MESH,
            )
        pl.semaphore_wait(barrier_sem, len(MASKS))

        out_ref[:, :] = x_ref[:, :]

        base = my * 0
        length = m_per
        bits = _keep_bits(my)
        for k, mask in enumerate(MASKS):
            half = length // 2
            partner = jnp.bitwise_xor(my, mask)
            b = bits[k]
            b_keep = base + b * half
            b_send = base + (1 - b) * half
            rdma = pltpu.make_async_remote_copy(
                src_ref=out_ref.at[pl.ds(b_send, half)],
                dst_ref=rs_bufs[k],
                send_sem=send_sems.at[k],
                recv_sem=recv_sems.at[k],
                device_id=(partner,),
                device_id_type=pl.DeviceIdType.MESH,
            )
            rdma.start()
            rdma.wait()
            out_ref[pl.ds(b_keep, half), :] = (
                out_ref[pl.ds(b_keep, half), :] + rs_bufs[k][:, :]
            )
            base = b_keep
            length = half


        for j in range(len(MASKS) - 1, -1, -1):
            mask = MASKS[j]
            partner = jnp.bitwise_xor(my, mask)
            rdma = pltpu.make_async_remote_copy(
                src_ref=out_ref.at[pl.ds(base, length)],
                dst_ref=out_ref.at[pl.ds(base, length)],
                send_sem=send_sems.at[4 + j],
                recv_sem=recv_sems.at[4 + j],
                device_id=(partner,),
                device_id_type=pl.DeviceIdType.MESH,
            )
            rdma.start()
            rdma.wait()
            base = base - bits[j] * length
            length = length * 2

    return pl.pallas_call(
        body,
        out_shape=jax.ShapeDtypeStruct((m_per, n), x.dtype),
        in_specs=[pl.BlockSpec(memory_space=pltpu.VMEM)],
        out_specs=pl.BlockSpec(memory_space=pltpu.VMEM),
        scratch_shapes=[
            pltpu.VMEM((m_per // 2, n), x.dtype),
            pltpu.VMEM((m_per // 4, n), x.dtype),
            pltpu.VMEM((m_per // 8, n), x.dtype),
            pltpu.VMEM((m_per // 16, n), x.dtype),
            pltpu.SemaphoreType.DMA((8,)),
            pltpu.SemaphoreType.DMA((8,)),
        ],
        compiler_params=pltpu.CompilerParams(collective_id=0),
    )(x)


# device time: 40061 ns/iter; 2.5861x vs baseline; 1.6140x over previous
import jax
import jax.numpy as jnp
from jax import lax
from jax.experimental import pallas as pl
from jax.experimental.pallas import tpu as pltpu

N_DEV = 16
N_RAILS = 4
N_STEPS = 4
RAIL_ORDERS = (
    (1, 3, 4, 8),
    (3, 4, 8, 1),
    (4, 8, 1, 3),
    (8, 1, 3, 4),
)


def _keep_bit(my, mask, later_masks):
    if mask == 1:
        if 3 in later_masks:
            return jnp.bitwise_xor(my, my >> 1) & 1
        return my & 1
    if mask == 3:
        return (my >> 1) & 1
    if mask == 4:
        return (my >> 2) & 1
    return (my >> 3) & 1


def kernel(x):
    m_per, n = x.shape
    cw = n // N_RAILS

    def body(x_ref, out_ref, rs0, rs1, rs2, rs3, send_sems, recv_sems):
        my = lax.axis_index("i")
        rs_bufs = [rs0, rs1, rs2, rs3]

        barrier_sem = pltpu.get_barrier_semaphore()
        for mask in (1, 3, 4, 8):
            pl.semaphore_signal(
                barrier_sem, inc=1,
                device_id=(jnp.bitwise_xor(my, mask),),
                device_id_type=pl.DeviceIdType.MESH,
            )
        pl.semaphore_wait(barrier_sem, 4)

        zero = my * 0
        bits = [
            [
                _keep_bit(my, m, RAIL_ORDERS[r][k + 1:])
                for k, m in enumerate(RAIL_ORDERS[r])
            ]
            for r in range(N_RAILS)
        ]
        base = [zero] * N_RAILS
        length = [m_per] * N_RAILS
        started = []

        def cols(r):
            return pl.ds(r * cw, cw)

        def rs_send(r, k):
            half = length[r] // 2
            b_send = base[r] + (1 - bits[r][k]) * half
            src_ref = x_ref if k == 0 else out_ref
            rdma = pltpu.make_async_remote_copy(
                src_ref=src_ref.at[pl.ds(b_send, half), cols(r)],
                dst_ref=rs_bufs[k].at[:, cols(r)],
                send_sem=send_sems.at[k * N_RAILS + r],
                recv_sem=recv_sems.at[k * N_RAILS + r],
                device_id=(jnp.bitwise_xor(my, RAIL_ORDERS[r][k]),),
                device_id_type=pl.DeviceIdType.MESH,
            )
            rdma.start()
            started.append(rdma)
            return rdma

        def rs_reduce(r, k):
            half = length[r] // 2
            b_keep = base[r] + bits[r][k] * half
            lhs = x_ref if k == 0 else out_ref
            out_ref[pl.ds(b_keep, half), cols(r)] = (
                lhs[pl.ds(b_keep, half), cols(r)] + rs_bufs[k][:, cols(r)]
            )
            base[r] = b_keep
            length[r] = half

        def ag_send(r, j):
            rdma = pltpu.make_async_remote_copy(
                src_ref=out_ref.at[pl.ds(base[r], length[r]), cols(r)],
                dst_ref=out_ref.at[pl.ds(base[r], length[r]), cols(r)],
                send_sem=send_sems.at[16 + j * N_RAILS + r],
                recv_sem=recv_sems.at[16 + j * N_RAILS + r],
                device_id=(jnp.bitwise_xor(my, RAIL_ORDERS[r][j]),),
                device_id_type=pl.DeviceIdType.MESH,
            )
            rdma.start()
            started.append(rdma)
            return rdma

        inflight = {}
        for r in range(N_RAILS):
            inflight[(r, 0)] = rs_send(r, 0)
        ag_inflight = {}
        for k in range(N_STEPS):
            for r in range(N_RAILS):
                inflight[(r, k)].wait_recv()
                rs_reduce(r, k)
                if k < N_STEPS - 1:
                    inflight[(r, k + 1)] = rs_send(r, k + 1)
                else:
                    ag_inflight[(r, N_STEPS - 1)] = ag_send(r, N_STEPS - 1)

        for j in range(N_STEPS - 1, -1, -1):
            for r in range(N_RAILS):
                ag_inflight[(r, j)].wait_recv()
                base[r] = base[r] - bits[r][j] * length[r]
                length[r] = length[r] * 2
                if j > 0:
                    ag_inflight[(r, j - 1)] = ag_send(r, j - 1)

        for rdma in started:
            rdma.wait_send()

    return pl.pallas_call(
        body,
        out_shape=jax.ShapeDtypeStruct((m_per, n), x.dtype),
        in_specs=[pl.BlockSpec(memory_space=pltpu.VMEM)],
        out_specs=pl.BlockSpec(memory_space=pltpu.VMEM),
        scratch_shapes=[
            pltpu.VMEM((m_per // 2, n), x.dtype),
            pltpu.VMEM((m_per // 4, n), x.dtype),
            pltpu.VMEM((m_per // 8, n), x.dtype),
            pltpu.VMEM((m_per // 16, n), x.dtype),
            pltpu.SemaphoreType.DMA((32,)),
            pltpu.SemaphoreType.DMA((32,)),
        ],
        compiler_params=pltpu.CompilerParams(collective_id=0),
    )(x)


# device time: 36101 ns/iter; 2.8697x vs baseline; 1.1097x over previous
import jax
import jax.numpy as jnp
from jax import lax
from jax.experimental import pallas as pl
from jax.experimental.pallas import tpu as pltpu

N_DEV = 16
N_RAILS = 4
N_STEPS = 4
N_SEMS = 64
RAIL_ORDERS = (
    (1, 3, 4, 8),
    (3, 4, 8, 1),
    (4, 8, 1, 3),
    (8, 1, 3, 4),
)


def _keep_bit(dev, mask, later_masks):
    if mask == 1:
        if 3 in later_masks:
            return jnp.bitwise_xor(dev, dev >> 1) & 1
        return dev & 1
    if mask == 3:
        return (dev >> 1) & 1
    if mask == 4:
        return (dev >> 2) & 1
    return (dev >> 3) & 1


def kernel(x):
    m_per, n = x.shape
    cw = n // N_RAILS

    def body(x_ref, out_ref, rs0, rs1, rs2, rs3, send_sems, recv_sems):
        my = lax.axis_index("i")
        rs_bufs = [rs0, rs1, rs2, rs3]

        barrier_sem = pltpu.get_barrier_semaphore()
        for mask in (1, 3, 4, 8):
            pl.semaphore_signal(
                barrier_sem, inc=1,
                device_id=(jnp.bitwise_xor(my, mask),),
                device_id_type=pl.DeviceIdType.MESH,
            )
        pl.semaphore_wait(barrier_sem, 4)

        bits = [
            [
                _keep_bit(my, m, RAIL_ORDERS[r][k + 1:])
                for k, m in enumerate(RAIL_ORDERS[r])
            ]
            for r in range(N_RAILS)
        ]
        partners = [
            [jnp.bitwise_xor(my, m) for m in RAIL_ORDERS[r]]
            for r in range(N_RAILS)
        ]
        base = [my * 0] * N_RAILS
        length = [m_per] * N_RAILS

        started = []
        sem_idx = [0]

        def cols(r):
            return pl.ds(r * cw, cw)

        def rcopy(src, dst, dev):
            i = sem_idx[0]
            sem_idx[0] += 1
            rdma = pltpu.make_async_remote_copy(
                src_ref=src,
                dst_ref=dst,
                send_sem=send_sems.at[i],
                recv_sem=recv_sems.at[i],
                device_id=(dev,),
                device_id_type=pl.DeviceIdType.MESH,
            )
            rdma.start()
            started.append(rdma)
            return rdma

        rs_handles = {}
        ag_handles = {}

        def rs_issue(r, k):
            order = RAIL_ORDERS[r]
            half = length[r] // 2
            q = partners[r][k]
            sb = base[r] + (1 - bits[r][k]) * half
            src = x_ref if k == 0 else out_ref
            if k < N_STEPS - 1:
                hh = half // 2
                bq = _keep_bit(q, order[k + 1], order[k + 2:])
                rel1 = (1 - bq) * hh
                rel2 = bq * hh
                d1 = rcopy(
                    src.at[pl.ds(sb + rel1, hh), cols(r)],
                    rs_bufs[k].at[pl.ds(rel1, hh), cols(r)], q)
                d2 = rcopy(
                    src.at[pl.ds(sb + rel2, hh), cols(r)],
                    rs_bufs[k].at[pl.ds(rel2, hh), cols(r)], q)
                rs_handles[(r, k)] = (d1, d2)
            else:
                d = rcopy(
                    src.at[pl.ds(sb, half), cols(r)],
                    rs_bufs[k].at[:, cols(r)], q)
                rs_handles[(r, k)] = (d,)

        def ag_send(r, j, start, ln):
            return rcopy(
                out_ref.at[pl.ds(start, ln), cols(r)],
                out_ref.at[pl.ds(start, ln), cols(r)],
                partners[r][j])

        def rs_consume(r, k):
            half = length[r] // 2
            kb = base[r] + bits[r][k] * half
            lhs = x_ref if k == 0 else out_ref
            if k < N_STEPS - 1:
                hh = half // 2
                bn = bits[r][k + 1]
                rel1 = (1 - bn) * hh
                rel2 = bn * hh
                d1, d2 = rs_handles[(r, k)]
                d1.wait_recv()
                out_ref[pl.ds(kb + rel1, hh), cols(r)] = (
                    lhs[pl.ds(kb + rel1, hh), cols(r)]
                    + rs_bufs[k][pl.ds(rel1, hh), cols(r)])
                base[r] = kb
                length[r] = half
                rs_issue(r, k + 1)
                d2.wait_recv()
                out_ref[pl.ds(kb + rel2, hh), cols(r)] = (
                    lhs[pl.ds(kb + rel2, hh), cols(r)]
                    + rs_bufs[k][pl.ds(rel2, hh), cols(r)])
            else:
                (d,) = rs_handles[(r, k)]
                d.wait_recv()
                out_ref[pl.ds(kb, half), cols(r)] = (
                    lhs[pl.ds(kb, half), cols(r)] + rs_bufs[k][:, cols(r)])
                base[r] = kb
                length[r] = half
                ag_handles[(r, 3)] = (ag_send(r, 3, base[r], half),)
                ag_handles[(r, 2, "A")] = ag_send(r, 2, base[r], half)

        for r in range(N_RAILS):
            rs_issue(r, 0)
        for k in range(N_STEPS):
            for r in range(N_RAILS):
                rs_consume(r, k)

        for j in range(N_STEPS - 1, -1, -1):
            for r in range(N_RAILS):
                if j == N_STEPS - 1:
                    handles = ag_handles[(r, 3)]
                else:
                    handles = (ag_handles[(r, j, "A")],
                               ag_handles[(r, j, "B")])
                for d in handles:
                    d.wait_recv()
                lenj = length[r]
                bj = bits[r][j]
                rstart = base[r] + (1 - 2 * bj) * lenj
                base[r] = base[r] - bj * lenj
                length[r] = 2 * lenj
                if j > 0:
                    ag_handles[(r, j - 1, "B")] = ag_send(
                        r, j - 1, rstart, lenj)
                if j > 1:
                    ag_handles[(r, j - 2, "A")] = ag_send(
                        r, j - 2, base[r], length[r])

        for rdma in started:
            rdma.wait_send()

    return pl.pallas_call(
        body,
        out_shape=jax.ShapeDtypeStruct((m_per, n), x.dtype),
        in_specs=[pl.BlockSpec(memory_space=pltpu.VMEM)],
        out_specs=pl.BlockSpec(memory_space=pltpu.VMEM),
        scratch_shapes=[
            pltpu.VMEM((m_per // 2, n), x.dtype),
            pltpu.VMEM((m_per // 4, n), x.dtype),
            pltpu.VMEM((m_per // 8, n), x.dtype),
            pltpu.VMEM((m_per // 16, n), x.dtype),
            pltpu.SemaphoreType.DMA((N_SEMS,)),
            pltpu.SemaphoreType.DMA((N_SEMS,)),
        ],
        compiler_params=pltpu.CompilerParams(collective_id=0),
    )(x)
